# baseline (speedup 1.0000x reference)
import functools

import jax
import jax.numpy as jnp
from jax import lax
from jax.experimental import pallas as pl
from jax.experimental.pallas import tpu as pltpu
from jax.experimental.pallas import tpu_sc as plsc

NC, NS, L = 2, 16, 16
NW = NC * NS


def _sc_gather(table, species2, moves2, ability2, status2, item2, offs, B):
    n_words = table.shape[0] * L
    bpw = species2.shape[1]
    out_words = bpw * 8 * L
    r_sp, r_mv, r_ab, r_st, r_it = 0, bpw, 5 * bpw, 6 * bpw, 7 * bpw

    mesh = plsc.VectorSubcoreMesh(core_axis_name="c", subcore_axis_name="s")

    @functools.partial(
        pl.kernel,
        mesh=mesh,
        out_type=jax.ShapeDtypeStruct((B * 8 * L,), jnp.float32),
        scratch_types=[
            pltpu.VMEM((n_words,), jnp.float32),
            pltpu.VMEM((8 * bpw,), jnp.int32),
            pltpu.VMEM((out_words,), jnp.float32),
        ],
        compiler_params=pltpu.CompilerParams(needs_layout_passes=False),
    )
    def body(table_hbm, sp_hbm, mv_hbm, ab_hbm, st_hbm, it_hbm, out_hbm,
             table_v, idx_v, rows_v):
        wid = lax.axis_index("s") * NC + lax.axis_index("c")
        pltpu.sync_copy(table_hbm, table_v)
        pltpu.sync_copy(sp_hbm.at[wid], idx_v.at[pl.ds(r_sp, bpw)])
        pltpu.sync_copy(mv_hbm.at[wid], idx_v.at[pl.ds(r_mv, 4 * bpw)])
        pltpu.sync_copy(ab_hbm.at[wid], idx_v.at[pl.ds(r_ab, bpw)])
        pltpu.sync_copy(st_hbm.at[wid], idx_v.at[pl.ds(r_st, bpw)])
        pltpu.sync_copy(it_hbm.at[wid], idx_v.at[pl.ds(r_it, bpw)])

        viota = jax.lax.iota(jnp.int32, L)

        def run_stream(region, count, off, dest_fn, unroll):
            @plsc.parallel_loop(0, count // L, unroll=unroll)
            def _step(j):
                p = viota + j * L
                rows = idx_v[pl.ds(region + j * L, L)] + off
                gbase = rows << 4
                sbase = dest_fn(p)
                for c in range(L):
                    vals = plsc.load_gather(table_v, [gbase + c])
                    plsc.store_scatter(rows_v, [sbase + c], vals)

        run_stream(r_sp, bpw, offs[0], lambda p: p << 7, 4)
        run_stream(r_mv, 4 * bpw, offs[1],
                   lambda p: ((p >> 2) << 7) + ((p & 3) << 4) + 16, 4)
        run_stream(r_ab, bpw, offs[2], lambda p: (p << 7) + 80, 4)
        run_stream(r_st, bpw, offs[3], lambda p: (p << 7) + 96, 4)
        run_stream(r_it, bpw, offs[4], lambda p: (p << 7) + 112, 4)

        pltpu.sync_copy(rows_v, out_hbm.at[pl.ds(wid * out_words, out_words)])

    return body(table.reshape(-1), species2, moves2, ability2, status2, item2)


def _mlp_body(g_ref, nt_ref, w1a_ref, w1b_ref, b1_ref, w2_ref, b2_ref, o_ref):
    h = jnp.dot(g_ref[...], w1a_ref[...], preferred_element_type=jnp.float32)
    h = h + lax.dot_general(
        nt_ref[...], w1b_ref[...], (((0,), (0,)), ((), ())),
        preferred_element_type=jnp.float32)
    h = jnp.maximum(h + b1_ref[...], 0.0)
    o_ref[...] = jnp.dot(h, w2_ref[...], preferred_element_type=jnp.float32) + b2_ref[...]


def kernel(species, moves, ability, status1, holdItem, hp, lvl, att, defn, spe,
           spA, spD, pp, exp, species_emb, move_emb, ability_emb, status_emb,
           item_emb, W1, b1, W2, b2):
    B = species.shape[0]
    f32, i32 = jnp.float32, jnp.int32

    o_m = species_emb.shape[0]
    o_a = o_m + move_emb.shape[0]
    o_st = o_a + ability_emb.shape[0]
    o_it = o_st + status_emb.shape[0]
    table = jnp.concatenate(
        [species_emb, move_emb, ability_emb, status_emb, item_emb], axis=0)
    pad = (-table.shape[0]) % 8
    if pad:
        table = jnp.concatenate([table, jnp.zeros((pad, L), f32)], axis=0)

    g = _sc_gather(
        table,
        species.astype(i32).reshape(NW, B // NW),
        moves.astype(i32).reshape(NW, 4 * B // NW),
        ability.astype(i32).reshape(NW, B // NW),
        status1.astype(i32).reshape(NW, B // NW),
        holdItem.astype(i32).reshape(NW, B // NW),
        (0, o_m, o_a, o_st, o_it),
        B,
    ).reshape(B, 8 * L)

    nt = jnp.concatenate([
        hp[None].astype(f32), lvl[None].astype(f32),
        att[None].astype(f32), defn[None].astype(f32),
        spe[None].astype(f32), spA[None].astype(f32), spD[None].astype(f32),
        jnp.mean(pp.astype(f32), axis=-1)[None], exp[None].astype(f32),
        jnp.zeros((7, B), f32),
    ], axis=0)
    w1a = W1[:128]
    w1b = jnp.concatenate([
        W1[128:129], W1[129:130] / 100.0, W1[130:137],
        jnp.zeros((7, 128), f32),
    ], axis=0)

    BLK = 2048
    out = pl.pallas_call(
        _mlp_body,
        grid=(B // BLK,),
        in_specs=[
            pl.BlockSpec((BLK, 128), lambda i: (i, 0)),
            pl.BlockSpec((16, BLK), lambda i: (0, i)),
            pl.BlockSpec((128, 128), lambda i: (0, 0)),
            pl.BlockSpec((16, 128), lambda i: (0, 0)),
            pl.BlockSpec((1, 128), lambda i: (0, 0)),
            pl.BlockSpec((128, 128), lambda i: (0, 0)),
            pl.BlockSpec((1, 128), lambda i: (0, 0)),
        ],
        out_specs=pl.BlockSpec((BLK, 128), lambda i: (i, 0)),
        out_shape=jax.ShapeDtypeStruct((B, 128), f32),
    )(g, nt, w1a, w1b, b1.reshape(1, 128), W2, b2.reshape(1, 128))
    return out


# speedup vs baseline: 1.1159x; 1.1159x over previous
import functools

import jax
import jax.numpy as jnp
from jax import lax
from jax.experimental import pallas as pl
from jax.experimental.pallas import tpu as pltpu
from jax.experimental.pallas import tpu_sc as plsc

NC, NS, L = 2, 16, 16
NW = NC * NS
RS = 17
OW = 8 * RS + 1


def _sc_gather(table, species, mv01, mv23, ability, status1, item, offs, B):
    n_words = table.shape[0]
    bpw = B // NW
    out_words = bpw * OW
    regions = [i * bpw for i in range(6)]

    mesh = plsc.VectorSubcoreMesh(core_axis_name="c", subcore_axis_name="s")

    @functools.partial(
        pl.kernel,
        mesh=mesh,
        out_type=jax.ShapeDtypeStruct((B * OW,), jnp.float32),
        scratch_types=[
            pltpu.VMEM((n_words,), jnp.float32),
            pltpu.VMEM((6 * bpw,), jnp.int32),
            pltpu.VMEM((out_words,), jnp.float32),
        ],
        compiler_params=pltpu.CompilerParams(needs_layout_passes=False),
    )
    def body(table_hbm, sp_hbm, m01_hbm, m23_hbm, ab_hbm, st_hbm, it_hbm,
             out_hbm, table_v, idx_v, rows_v):
        wid = lax.axis_index("s") * NC + lax.axis_index("c")
        base = wid * bpw
        pltpu.sync_copy(table_hbm, table_v)
        for reg, src in zip(regions,
                            (sp_hbm, m01_hbm, m23_hbm, ab_hbm, st_hbm, it_hbm)):
            pltpu.sync_copy(src.at[pl.ds(base, bpw)],
                            idx_v.at[pl.ds(reg, bpw)])

        viota = jax.lax.iota(jnp.int32, L)
        vzero = jnp.zeros((L,), jnp.float32)

        def emit_slot(rows, sbase):
            gbase = rows * RS
            for c in range(RS):
                vals = plsc.load_gather(table_v, [gbase + c])
                plsc.store_scatter(rows_v, [sbase + c], vals)

        @plsc.parallel_loop(0, bpw // L, unroll=4)
        def _sp(j):
            p = viota + j * L
            rows = idx_v[pl.ds(regions[0] + j * L, L)] + offs[0]
            emit_slot(rows, p * OW)
            plsc.store_scatter(rows_v, [p * OW + (OW - 1)], vzero)

        for reg, slots in ((regions[1], (1, 2)), (regions[2], (3, 4))):
            @plsc.parallel_loop(0, bpw // L, unroll=4)
            def _mv(j, reg=reg, slots=slots):
                p = viota + j * L
                packed = idx_v[pl.ds(reg + j * L, L)]
                lo = (packed & 0xFFFF) + offs[1]
                hi = lax.shift_right_logical(packed, 16) + offs[1]
                emit_slot(lo, p * OW + slots[0] * RS)
                emit_slot(hi, p * OW + slots[1] * RS)

        for reg, off, slot in ((regions[3], offs[2], 5),
                               (regions[4], offs[3], 6),
                               (regions[5], offs[4], 7)):
            @plsc.parallel_loop(0, bpw // L, unroll=4)
            def _single(j, reg=reg, off=off, slot=slot):
                p = viota + j * L
                rows = idx_v[pl.ds(reg + j * L, L)] + off
                emit_slot(rows, p * OW + slot * RS)

        pltpu.sync_copy(rows_v, out_hbm.at[pl.ds(wid * out_words, out_words)])

    return body(table, species, mv01, mv23, ability, status1, item)


def _mlp_body(g_ref, nt_ref, w1p_ref, w1b_ref, b1_ref, w2_ref, b2_ref, o_ref):
    h = jnp.dot(g_ref[...], w1p_ref[...], preferred_element_type=jnp.float32)
    h = h + lax.dot_general(
        nt_ref[...], w1b_ref[...], (((0,), (0,)), ((), ())),
        preferred_element_type=jnp.float32)
    h = jnp.maximum(h + b1_ref[...], 0.0)
    o_ref[...] = jnp.dot(h, w2_ref[...], preferred_element_type=jnp.float32) + b2_ref[...]


def kernel(species, moves, ability, status1, holdItem, hp, lvl, att, defn, spe,
           spA, spD, pp, exp, species_emb, move_emb, ability_emb, status_emb,
           item_emb, W1, b1, W2, b2):
    B = species.shape[0]
    f32, i32 = jnp.float32, jnp.int32

    o_m = species_emb.shape[0]
    o_a = o_m + move_emb.shape[0]
    o_st = o_a + ability_emb.shape[0]
    o_it = o_st + status_emb.shape[0]
    table = jnp.concatenate(
        [species_emb, move_emb, ability_emb, status_emb, item_emb], axis=0)
    table = jnp.pad(table, ((0, (-table.shape[0]) % 8), (0, RS - L)))

    mv = moves.astype(i32)
    mv01 = mv[:, 0] | (mv[:, 1] << 16)
    mv23 = mv[:, 2] | (mv[:, 3] << 16)

    g = _sc_gather(
        table.reshape(-1), species.astype(i32), mv01, mv23,
        ability.astype(i32), status1.astype(i32), holdItem.astype(i32),
        (0, o_m, o_a, o_st, o_it), B,
    ).reshape(B, OW)

    nt = jnp.concatenate([
        hp[None].astype(f32), lvl[None].astype(f32),
        att[None].astype(f32), defn[None].astype(f32),
        spe[None].astype(f32), spA[None].astype(f32), spD[None].astype(f32),
        jnp.mean(pp.astype(f32), axis=-1)[None], exp[None].astype(f32),
        jnp.zeros((7, B), f32),
    ], axis=0)
    w1p = jnp.pad(W1[:128].reshape(8, L, 128),
                  ((0, 0), (0, RS - L), (0, 0))).reshape(8 * RS, 128)
    w1p = jnp.pad(w1p, ((0, 1), (0, 0)))
    w1b = jnp.concatenate([
        W1[128:129], W1[129:130] / 100.0, W1[130:137],
        jnp.zeros((7, 128), f32),
    ], axis=0)

    BLK = 2048
    out = pl.pallas_call(
        _mlp_body,
        grid=(B // BLK,),
        in_specs=[
            pl.BlockSpec((BLK, OW), lambda i: (i, 0)),
            pl.BlockSpec((16, BLK), lambda i: (0, i)),
            pl.BlockSpec((OW, 128), lambda i: (0, 0)),
            pl.BlockSpec((16, 128), lambda i: (0, 0)),
            pl.BlockSpec((1, 128), lambda i: (0, 0)),
            pl.BlockSpec((128, 128), lambda i: (0, 0)),
            pl.BlockSpec((1, 128), lambda i: (0, 0)),
        ],
        out_specs=pl.BlockSpec((BLK, 128), lambda i: (i, 0)),
        out_shape=jax.ShapeDtypeStruct((B, 128), f32),
    )(g, nt, w1p, w1b, b1.reshape(1, 128), W2, b2.reshape(1, 128))
    return out


# speedup vs baseline: 1.7346x; 1.5544x over previous
import functools

import jax
import jax.numpy as jnp
from jax import lax
from jax.experimental import pallas as pl
from jax.experimental.pallas import tpu as pltpu
from jax.experimental.pallas import tpu_sc as plsc

NC, NS, L = 2, 16, 16
NW = NC * NS
RS = 17


def _sc_gather(table, species, mv01, mv23, ability, status1, item, offs, B):
    n_words = table.shape[0]
    bpw = B // NW
    regions = [i * bpw for i in range(6)]

    mesh = plsc.VectorSubcoreMesh(core_axis_name="c", subcore_axis_name="s")

    @functools.partial(
        pl.kernel,
        mesh=mesh,
        out_type=jax.ShapeDtypeStruct((8 * L, B), jnp.float32),
        scratch_types=[
            pltpu.VMEM((n_words,), jnp.float32),
            pltpu.VMEM((6 * bpw,), jnp.int32),
            pltpu.VMEM((8 * L, bpw), jnp.float32),
        ],
        compiler_params=pltpu.CompilerParams(needs_layout_passes=False),
    )
    def body(table_hbm, sp_hbm, m01_hbm, m23_hbm, ab_hbm, st_hbm, it_hbm,
             out_hbm, table_v, idx_v, rows_v):
        wid = lax.axis_index("s") * NC + lax.axis_index("c")
        base = wid * bpw
        pltpu.sync_copy(table_hbm, table_v)
        for reg, src in zip(regions,
                            (sp_hbm, m01_hbm, m23_hbm, ab_hbm, st_hbm, it_hbm)):
            pltpu.sync_copy(src.at[pl.ds(base, bpw)],
                            idx_v.at[pl.ds(reg, bpw)])

        def emit_slot(rows, slot, j):
            gbase = rows * RS
            for c in range(L):
                vals = plsc.load_gather(table_v, [gbase + c])
                rows_v[slot * L + c, pl.ds(j * L, L)] = vals

        @plsc.parallel_loop(0, bpw // L, unroll=4)
        def _sp(j):
            rows = idx_v[pl.ds(regions[0] + j * L, L)] + offs[0]
            emit_slot(rows, 0, j)

        for reg, slots in ((regions[1], (1, 2)), (regions[2], (3, 4))):
            @plsc.parallel_loop(0, bpw // L, unroll=4)
            def _mv(j, reg=reg, slots=slots):
                packed = idx_v[pl.ds(reg + j * L, L)]
                emit_slot((packed & 0xFFFF) + offs[1], slots[0], j)
                emit_slot(lax.shift_right_logical(packed, 16) + offs[1],
                          slots[1], j)

        for reg, off, slot in ((regions[3], offs[2], 5),
                               (regions[4], offs[3], 6),
                               (regions[5], offs[4], 7)):
            @plsc.parallel_loop(0, bpw // L, unroll=4)
            def _single(j, reg=reg, off=off, slot=slot):
                rows = idx_v[pl.ds(reg + j * L, L)] + off
                emit_slot(rows, slot, j)

        pltpu.sync_copy(rows_v, out_hbm.at[:, pl.ds(base, bpw)])

    return body(table, species, mv01, mv23, ability, status1, item)


def _mlp_body(gt_ref, nt_ref, w1a_ref, w1b_ref, b1_ref, w2_ref, b2_ref, o_ref):
    h = lax.dot_general(
        gt_ref[...], w1a_ref[...], (((0,), (0,)), ((), ())),
        preferred_element_type=jnp.float32)
    h = h + lax.dot_general(
        nt_ref[...], w1b_ref[...], (((0,), (0,)), ((), ())),
        preferred_element_type=jnp.float32)
    h = jnp.maximum(h + b1_ref[...], 0.0)
    o_ref[...] = jnp.dot(h, w2_ref[...], preferred_element_type=jnp.float32) + b2_ref[...]


def kernel(species, moves, ability, status1, holdItem, hp, lvl, att, defn, spe,
           spA, spD, pp, exp, species_emb, move_emb, ability_emb, status_emb,
           item_emb, W1, b1, W2, b2):
    B = species.shape[0]
    f32, i32 = jnp.float32, jnp.int32

    o_m = species_emb.shape[0]
    o_a = o_m + move_emb.shape[0]
    o_st = o_a + ability_emb.shape[0]
    o_it = o_st + status_emb.shape[0]
    table = jnp.concatenate(
        [species_emb, move_emb, ability_emb, status_emb, item_emb], axis=0)
    table = jnp.pad(table, ((0, (-table.shape[0]) % 8), (0, RS - L)))

    mv = moves.astype(i32)
    mv01 = mv[:, 0] | (mv[:, 1] << 16)
    mv23 = mv[:, 2] | (mv[:, 3] << 16)

    gt = _sc_gather(
        table.reshape(-1), species.astype(i32), mv01, mv23,
        ability.astype(i32), status1.astype(i32), holdItem.astype(i32),
        (0, o_m, o_a, o_st, o_it), B,
    )

    nt = jnp.concatenate([
        hp[None].astype(f32), lvl[None].astype(f32),
        att[None].astype(f32), defn[None].astype(f32),
        spe[None].astype(f32), spA[None].astype(f32), spD[None].astype(f32),
        jnp.mean(pp.astype(f32), axis=-1)[None], exp[None].astype(f32),
        jnp.zeros((7, B), f32),
    ], axis=0)
    w1a = W1[:128]
    w1b = jnp.concatenate([
        W1[128:129], W1[129:130] / 100.0, W1[130:137],
        jnp.zeros((7, 128), f32),
    ], axis=0)

    BLK = 2048
    out = pl.pallas_call(
        _mlp_body,
        grid=(B // BLK,),
        in_specs=[
            pl.BlockSpec((128, BLK), lambda i: (0, i)),
            pl.BlockSpec((16, BLK), lambda i: (0, i)),
            pl.BlockSpec((128, 128), lambda i: (0, 0)),
            pl.BlockSpec((16, 128), lambda i: (0, 0)),
            pl.BlockSpec((1, 128), lambda i: (0, 0)),
            pl.BlockSpec((128, 128), lambda i: (0, 0)),
            pl.BlockSpec((1, 128), lambda i: (0, 0)),
        ],
        out_specs=pl.BlockSpec((BLK, 128), lambda i: (i, 0)),
        out_shape=jax.ShapeDtypeStruct((B, 128), f32),
    )(gt, nt, w1a, w1b, b1.reshape(1, 128), W2, b2.reshape(1, 128))
    return out
